# Initial kernel scaffold; baseline (speedup 1.0000x reference)
import functools

import jax
import jax.numpy as jnp
from jax import lax
from jax.experimental import pallas as pl
from jax.experimental.pallas import tpu as pltpu
from jax.experimental.pallas import tpu_sc as plsc

B = 16
T_OUT = 375
V = 16384
D = 1280
TOTAL_TOK = 6000

SLOT = 384
HALF = SLOT // 2
CH = 64
NCH = HALF // CH
FT_PAD = 6400


def _sc_gather(ft_pad, cu_pad, codebook):
    mesh = plsc.VectorSubcoreMesh(core_axis_name="c", subcore_axis_name="s")

    @functools.partial(
        pl.kernel,
        mesh=mesh,
        out_type=jax.ShapeDtypeStruct((B * SLOT, D), jnp.float32),
        scratch_types=[
            pltpu.VMEM((FT_PAD,), jnp.int32),
            pltpu.VMEM((32,), jnp.int32),
            pltpu.VMEM((CH,), jnp.int32),
            pltpu.VMEM((CH, D), jnp.float32),
            pltpu.SemaphoreType.DMA,
        ],
    )
    def sck(ft_hbm, cu_hbm, cb_hbm, out_hbm, ftbuf, cubuf, idxbuf, rowsbuf, sem):
        wid = lax.axis_index("s") * 2 + lax.axis_index("c")
        b = wid // 2
        h = wid % 2

        pltpu.sync_copy(ft_hbm, ftbuf)
        pltpu.sync_copy(cu_hbm, cubuf)

        lanes = lax.iota(jnp.int32, 16)
        cuv = cubuf[pl.ds(0, 16)]
        cu_b = lax.reduce_max(jnp.where(lanes == b, cuv, -1), axes=(0,))

        base = cu_b + h * HALF
        row0 = b * SLOT + h * HALF

        for j in range(NCH):
            for c in range(CH // 16):
                idxbuf[pl.ds(c * 16, 16)] = ftbuf[pl.ds(base + j * CH + c * 16, 16)]
            pltpu.async_copy(cb_hbm.at[idxbuf], rowsbuf, sem).wait()
            pltpu.sync_copy(rowsbuf, out_hbm.at[pl.ds(row0 + j * CH, CH)])

    return sck(ft_pad, cu_pad, codebook)


def _tc_body(cu_ref, g_ref, out_ref, len_ref):
    b = pl.program_id(0)
    l_raw = cu_ref[b + 1] - cu_ref[b]
    len_ref[b] = l_raw
    l_eff = jnp.minimum(l_raw, T_OUT)
    x = g_ref[0]
    xt = jnp.transpose(x)
    tcol = lax.broadcasted_iota(jnp.int32, (D, T_OUT), 1)
    out_ref[0] = jnp.where(tcol < l_eff, xt[:, :T_OUT], 0.0)


def _tc_transpose(gathered, cu):
    g3 = gathered.reshape(B, SLOT, D)
    return pl.pallas_call(
        _tc_body,
        grid=(B,),
        in_specs=[
            pl.BlockSpec(memory_space=pltpu.SMEM),
            pl.BlockSpec((1, SLOT, D), lambda b: (b, 0, 0)),
        ],
        out_specs=[
            pl.BlockSpec((1, D, T_OUT), lambda b: (b, 0, 0)),
            pl.BlockSpec(memory_space=pltpu.SMEM, index_map=lambda b: (0,)),
        ],
        out_shape=[
            jax.ShapeDtypeStruct((B, D, T_OUT), jnp.float32),
            jax.ShapeDtypeStruct((B,), jnp.int32),
        ],
    )(cu, g3)


@jax.jit
def kernel(flat_tokens, cu_seqlens, codebook):
    ft = flat_tokens.astype(jnp.int32)
    cu = cu_seqlens.astype(jnp.int32)
    ft_pad = jnp.concatenate([ft, jnp.zeros((FT_PAD - TOTAL_TOK,), jnp.int32)])
    cu_pad = jnp.concatenate([cu, jnp.zeros((32 - (B + 1),), jnp.int32)])
    gathered = _sc_gather(ft_pad, cu_pad, codebook)
    out, lengths = _tc_transpose(gathered, cu)
    return out, lengths


# speedup vs baseline: 3.1468x; 3.1468x over previous
import dataclasses
import functools

import jax
import jax.numpy as jnp
from jax import lax
from jax.experimental import pallas as pl
from jax.experimental.pallas import tpu as pltpu
from jax.experimental.pallas import tpu_sc as plsc

B = 16
T_OUT = 375
V = 16384
D = 1280
TOTAL_TOK = 6000

SLOT = 384
HALF = SLOT // 2
CH = 64
NCH = HALF // CH
FT_PAD = 6400


def _sc_gather(ft_pad, cu_pad, codebook):
    mesh = plsc.VectorSubcoreMesh(core_axis_name="c", subcore_axis_name="s")
    cp = pltpu.CompilerParams()
    if "needs_layout_passes" in pltpu.CompilerParams.__dataclass_fields__:
        cp = dataclasses.replace(cp, needs_layout_passes=False)

    @functools.partial(
        pl.kernel,
        mesh=mesh,
        compiler_params=cp,
        out_type=jax.ShapeDtypeStruct((B * SLOT, D), jnp.float32),
        scratch_types=[
            pltpu.VMEM((FT_PAD,), jnp.int32),
            pltpu.VMEM((32,), jnp.int32),
            pltpu.VMEM((CH,), jnp.int32),
            pltpu.VMEM((CH, D), jnp.float32),
            pltpu.SemaphoreType.DMA,
        ],
    )
    def sck(ft_hbm, cu_hbm, cb_hbm, out_hbm, ftbuf, cubuf, idxbuf, rowsbuf, sem):
        wid = lax.axis_index("s") * 2 + lax.axis_index("c")
        b = wid // 2
        h = wid % 2

        pltpu.sync_copy(ft_hbm, ftbuf)
        pltpu.sync_copy(cu_hbm, cubuf)

        lanes = lax.iota(jnp.int32, 16)
        bsplat = jnp.full((16,), 0, jnp.int32) + b
        cu_b_v = plsc.load_gather(cubuf, [bsplat])

        row0 = b * SLOT + h * HALF

        for j in range(NCH):
            for c in range(CH // 16):
                posv = cu_b_v + (h * HALF + j * CH + c * 16) + lanes
                idxbuf[pl.ds(c * 16, 16)] = plsc.load_gather(ftbuf, [posv])
            pltpu.async_copy(cb_hbm.at[idxbuf], rowsbuf, sem).wait()
            pltpu.sync_copy(rowsbuf, out_hbm.at[pl.ds(row0 + j * CH, CH)])

    return sck(ft_pad, cu_pad, codebook)


def _tc_body(cu_ref, g_ref, out_ref, len_ref):
    b = pl.program_id(0)
    l_raw = cu_ref[b + 1] - cu_ref[b]
    len_ref[b] = l_raw
    l_eff = jnp.minimum(l_raw, T_OUT)
    x = g_ref[0]
    xt = jnp.transpose(x)
    tcol = lax.broadcasted_iota(jnp.int32, (D, T_OUT), 1)
    out_ref[0] = jnp.where(tcol < l_eff, xt[:, :T_OUT], 0.0)


def _tc_transpose(gathered, cu):
    g3 = gathered.reshape(B, SLOT, D)
    return pl.pallas_call(
        _tc_body,
        grid=(B,),
        in_specs=[
            pl.BlockSpec(memory_space=pltpu.SMEM),
            pl.BlockSpec((1, SLOT, D), lambda b: (b, 0, 0)),
        ],
        out_specs=[
            pl.BlockSpec((1, D, T_OUT), lambda b: (b, 0, 0)),
            pl.BlockSpec(memory_space=pltpu.SMEM, index_map=lambda b: (0,)),
        ],
        out_shape=[
            jax.ShapeDtypeStruct((B, D, T_OUT), jnp.float32),
            jax.ShapeDtypeStruct((B,), jnp.int32),
        ],
    )(cu, g3)


@jax.jit
def kernel(flat_tokens, cu_seqlens, codebook):
    ft = flat_tokens.astype(jnp.int32)
    cu = cu_seqlens.astype(jnp.int32)
    ft_pad = jnp.concatenate([ft, jnp.zeros((FT_PAD - TOTAL_TOK,), jnp.int32)])
    cu_pad = jnp.concatenate([cu, jnp.zeros((32 - (B + 1),), jnp.int32)])
    gathered = _sc_gather(ft_pad, cu_pad, codebook)
    out, lengths = _tc_transpose(gathered, cu)
    return out, lengths


# speedup vs baseline: 3.2490x; 1.0325x over previous
import dataclasses
import functools

import jax
import jax.numpy as jnp
from jax import lax
from jax.experimental import pallas as pl
from jax.experimental.pallas import tpu as pltpu
from jax.experimental.pallas import tpu_sc as plsc

B = 16
T_OUT = 375
V = 16384
D = 1280
TOTAL_TOK = 6000

SLOT = 384
HALF = SLOT // 2
CH = 32
NCH = HALF // CH


def _sc_gather(flat_tokens, cu, codebook):
    mesh = plsc.VectorSubcoreMesh(core_axis_name="c", subcore_axis_name="s")
    cp = pltpu.CompilerParams()
    if "needs_layout_passes" in pltpu.CompilerParams.__dataclass_fields__:
        cp = dataclasses.replace(cp, needs_layout_passes=False)

    @functools.partial(
        pl.kernel,
        mesh=mesh,
        compiler_params=cp,
        out_type=jax.ShapeDtypeStruct((B * SLOT, D), jnp.float32),
        scratch_types=[
            pltpu.VMEM((TOTAL_TOK,), jnp.int32),
            pltpu.VMEM((32,), jnp.int32),
            pltpu.VMEM((CH,), jnp.int32),
            pltpu.VMEM((CH,), jnp.int32),
            pltpu.VMEM((CH, D), jnp.float32),
            pltpu.VMEM((CH, D), jnp.float32),
            pltpu.SemaphoreType.DMA,
            pltpu.SemaphoreType.DMA,
        ],
    )
    def sck(ft_hbm, cu_hbm, cb_hbm, out_hbm,
            ftbuf, cubuf, idx0, idx1, rows0, rows1, gsem, wsem):
        wid = lax.axis_index("s") * 2 + lax.axis_index("c")
        b = wid // 2
        h = wid % 2

        pltpu.sync_copy(ft_hbm, ftbuf)
        pltpu.sync_copy(cu_hbm, cubuf.at[pl.ds(0, B + 1)])

        lanes = lax.iota(jnp.int32, 16)
        bsplat = jnp.full((16,), 0, jnp.int32) + b
        cu_b_v = plsc.load_gather(cubuf, [bsplat])

        row0 = b * SLOT + h * HALF
        idxs = (idx0, idx1)
        rows = (rows0, rows1)

        def fill_idx(buf, j):
            for c in range(CH // 16):
                posv = cu_b_v + (h * HALF + j * CH + c * 16) + lanes
                posv = jnp.minimum(posv, TOTAL_TOK - 1)
                buf[pl.ds(c * 16, 16)] = plsc.load_gather(ftbuf, [posv])

        gathers = {}
        writes = {}
        fill_idx(idx0, 0)
        gathers[0] = pltpu.async_copy(cb_hbm.at[idx0], rows0, gsem)
        for j in range(NCH):
            pb = j % 2
            nb = 1 - pb
            if j >= 1:
                writes[j - 1].wait()
            if j + 1 < NCH:
                fill_idx(idxs[nb], j + 1)
                gathers[j + 1] = pltpu.async_copy(cb_hbm.at[idxs[nb]], rows[nb], gsem)
            gathers[j].wait()
            writes[j] = pltpu.async_copy(
                rows[pb], out_hbm.at[pl.ds(row0 + j * CH, CH)], wsem)
        writes[NCH - 1].wait()

    return sck(flat_tokens, cu, codebook)


def _tc_body(cu_ref, g_ref, out_ref, len_ref):
    b = pl.program_id(0)
    l_raw = cu_ref[b + 1] - cu_ref[b]
    len_ref[b] = l_raw
    l_eff = jnp.minimum(l_raw, T_OUT)
    x = g_ref[0]
    xt = jnp.transpose(x)
    tcol = lax.broadcasted_iota(jnp.int32, (D, T_OUT), 1)
    out_ref[0] = jnp.where(tcol < l_eff, xt[:, :T_OUT], 0.0)


def _tc_transpose(gathered, cu):
    g3 = gathered.reshape(B, SLOT, D)
    return pl.pallas_call(
        _tc_body,
        grid=(B,),
        in_specs=[
            pl.BlockSpec(memory_space=pltpu.SMEM),
            pl.BlockSpec((1, SLOT, D), lambda b: (b, 0, 0)),
        ],
        out_specs=[
            pl.BlockSpec((1, D, T_OUT), lambda b: (b, 0, 0)),
            pl.BlockSpec(memory_space=pltpu.SMEM, index_map=lambda b: (0,)),
        ],
        out_shape=[
            jax.ShapeDtypeStruct((B, D, T_OUT), jnp.float32),
            jax.ShapeDtypeStruct((B,), jnp.int32),
        ],
    )(cu, g3)


@jax.jit
def kernel(flat_tokens, cu_seqlens, codebook):
    ft = flat_tokens.astype(jnp.int32)
    cu = cu_seqlens.astype(jnp.int32)
    gathered = _sc_gather(ft, cu, codebook)
    out, lengths = _tc_transpose(gathered, cu)
    return out, lengths
